# baseline (speedup 1.0000x reference)
import dataclasses
import functools

import numpy as np
import jax
import jax.numpy as jnp
from jax import lax
from jax.experimental import pallas as pl
from jax.experimental.pallas import tpu as pltpu
from jax.experimental.pallas import tpu_sc as plsc

VOCAB = 1000000
EMB = 64
SEQ = 50
BATCH = 16384
HID = 256
NCLS = 3

NC = 2
NS = 16
NW = NC * NS

NROWS = 2 * BATCH
NPAIRS = NROWS // 2
PAIRS_PER_W = NPAIRS // NW
CP = 16
NCH = PAIRS_PER_W // CP

_INV_SEQ = 1.0 / SEQ

_PERM = np.concatenate(
    [np.concatenate([q * 32 + np.arange(0, 32, 2),
                     q * 32 + np.arange(1, 32, 2)]) for q in range(EMB // 32)]
)


def _sc_compiler_params():
    cp = pltpu.CompilerParams(use_tc_tiling_on_sc=False)
    if "needs_layout_passes" in pltpu.CompilerParams.__dataclass_fields__:
        cp = dataclasses.replace(cp, needs_layout_passes=False)
    return cp


def _pool_sc(table_dup, idx_pairs):
    mesh = plsc.VectorSubcoreMesh(core_axis_name="c", subcore_axis_name="s")

    @functools.partial(
        pl.kernel,
        mesh=mesh,
        out_type=jax.ShapeDtypeStruct((NROWS, EMB), jnp.float32),
        compiler_params=_sc_compiler_params(),
        scratch_types=[
            pltpu.VMEM((CP, 2 * SEQ), jnp.int32),
            pltpu.VMEM((2, 2 * SEQ, 2 * EMB), jnp.bfloat16),
            pltpu.VMEM((2 * CP, EMB), jnp.float32),
            pltpu.SemaphoreType.DMA,
            pltpu.SemaphoreType.DMA,
        ],
    )
    def k(table_hbm, idx_hbm, out_hbm, idx_v, rows_v, out_v, sem0, sem1):
        wid = lax.axis_index("s") * NC + lax.axis_index("c")
        base_pair = wid * PAIRS_PER_W
        sems = (sem0, sem1)
        unpack = functools.partial(plsc.unpack,
                                   format=plsc.PackFormat.INTERLEAVED)

        def reduce_pair(buf, p):
            for half in range(2):
                r0 = half * SEQ
                for q in range(EMB // 32):
                    qo = q * 32

                    def ld(j):
                        return rows_v[buf, r0 + j, pl.ds(qo, 32)]

                    e0, o0 = unpack(ld(0))
                    e1, o1 = unpack(ld(1))
                    for j in range(2, SEQ, 2):
                        ea, oa = unpack(ld(j))
                        e0 = e0 + ea
                        o0 = o0 + oa
                        eb, ob = unpack(ld(j + 1))
                        e1 = e1 + eb
                        o1 = o1 + ob
                    out_v[2 * p + half, pl.ds(qo, 16)] = \
                        (e0 + e1) * _INV_SEQ
                    out_v[2 * p + half, pl.ds(qo + 16, 16)] = \
                        (o0 + o1) * _INV_SEQ

        def gather(p, buf):
            pltpu.async_copy(table_hbm.at[idx_v.at[p]], rows_v.at[buf],
                             sems[buf])

        def wait(buf):
            pltpu.make_async_copy(table_hbm.at[idx_v.at[0]], rows_v.at[buf],
                                  sems[buf]).wait()

        @pl.loop(0, NCH)
        def _(ch):
            pair0 = base_pair + ch * CP
            pltpu.sync_copy(idx_hbm.at[pl.ds(pair0, CP), :], idx_v)
            gather(0, 0)

            @pl.loop(0, CP, step=2)
            def _(p):
                gather(p + 1, 1)
                wait(0)
                reduce_pair(0, p)

                @pl.when(p + 2 < CP)
                def _():
                    gather(p + 2, 0)

                wait(1)
                reduce_pair(1, p + 1)

            pltpu.sync_copy(out_v, out_hbm.at[pl.ds(2 * pair0, 2 * CP), :])

    return k(table_dup, idx_pairs)


_BB = 2048
_NB = BATCH // _BB


def _mlp_body(xp_ref, xh_ref, w1p_ref, w1h_ref, b1_ref, w2_ref, b2_ref,
              w3_ref, b3_ref, wp_ref, bp_ref, o_ref):
    dot = functools.partial(jnp.dot, preferred_element_type=jnp.float32,
                            precision=lax.Precision.HIGHEST)
    h = dot(xp_ref[...], w1p_ref[...])
    h = h + dot(xh_ref[...], w1h_ref[...]) + b1_ref[...]
    h = jnp.maximum(h, 0.0)
    h = jnp.maximum(dot(h, w2_ref[...]) + b2_ref[...], 0.0)
    h = jnp.maximum(dot(h, w3_ref[...]) + b3_ref[...], 0.0)
    o_ref[...] = dot(h, wp_ref[...]) + bp_ref[...]


def _mlp_tc(pooled, W1p, W1h, b1, W2, b2, W3, b3, Wp, bp):
    full = lambda shape: pl.BlockSpec(shape, lambda i: (0, 0))
    return pl.pallas_call(
        _mlp_body,
        grid=(_NB,),
        in_specs=[
            pl.BlockSpec((_BB, EMB), lambda i: (i, 0)),
            pl.BlockSpec((_BB, EMB), lambda i: (i + _NB, 0)),
            full((EMB, HID)),
            full((EMB, HID)),
            full((1, HID)),
            full((HID, HID)),
            full((1, HID)),
            full((HID, HID)),
            full((1, HID)),
            full((HID, NCLS)),
            full((1, NCLS)),
        ],
        out_specs=pl.BlockSpec((_BB, NCLS), lambda i: (i, 0)),
        out_shape=jax.ShapeDtypeStruct((BATCH, NCLS), jnp.float32),
    )(pooled, pooled, W1p, W1h, b1, W2, b2, W3, b3, Wp, bp)


def kernel(premise, hypothesis, table, W1, b1, W2, b2, W3, b3, Wp, bp):
    tb = table.astype(jnp.bfloat16)
    table_dup = jnp.concatenate(
        [tb, jnp.concatenate([tb[1:], tb[:1]], axis=0)], axis=1)
    idx_pairs = jnp.concatenate([premise, hypothesis], axis=0)
    idx_pairs = idx_pairs.reshape(NPAIRS, 2 * SEQ)
    pooled = _pool_sc(table_dup, idx_pairs)
    W1perm = W1[jnp.concatenate([jnp.asarray(_PERM),
                                 jnp.asarray(_PERM) + EMB])]
    return _mlp_tc(pooled, W1perm[:EMB], W1perm[EMB:], b1.reshape(1, HID),
                   W2, b2.reshape(1, HID), W3, b3.reshape(1, HID),
                   Wp, bp.reshape(1, NCLS))


# speedup vs baseline: 1.8496x; 1.8496x over previous
import dataclasses
import functools

import numpy as np
import jax
import jax.numpy as jnp
from jax import lax
from jax.experimental import pallas as pl
from jax.experimental.pallas import tpu as pltpu
from jax.experimental.pallas import tpu_sc as plsc

VOCAB = 1000000
EMB = 64
SEQ = 50
BATCH = 16384
HID = 256
NCLS = 3

NC = 2
NS = 16
NW = NC * NS

NROWS = 2 * BATCH
NPAIRS = NROWS // 2
PAIRS_PER_W = NPAIRS // NW
CP = 16
NCH = PAIRS_PER_W // CP

_INV_SEQ = 1.0 / SEQ

_PERM = np.concatenate(
    [np.concatenate([q * 32 + np.arange(0, 32, 2),
                     q * 32 + np.arange(1, 32, 2)]) for q in range(EMB // 32)]
)


def _sc_compiler_params():
    cp = pltpu.CompilerParams(use_tc_tiling_on_sc=True)
    if "needs_layout_passes" in pltpu.CompilerParams.__dataclass_fields__:
        cp = dataclasses.replace(cp, needs_layout_passes=False)
    return cp


def _pool_sc(table_pad, idx_pairs):
    mesh = plsc.VectorSubcoreMesh(core_axis_name="c", subcore_axis_name="s")

    @functools.partial(
        pl.kernel,
        mesh=mesh,
        out_type=jax.ShapeDtypeStruct((NROWS, EMB), jnp.float32),
        compiler_params=_sc_compiler_params(),
        scratch_types=[
            pltpu.VMEM((CP, 2 * SEQ), jnp.int32),
            pltpu.VMEM((2 * SEQ, 2 * EMB), jnp.float32),
            pltpu.VMEM((2 * SEQ, 2 * EMB), jnp.float32),
            pltpu.VMEM((2 * CP, EMB), jnp.float32),
            pltpu.SemaphoreType.DMA,
            pltpu.SemaphoreType.DMA,
        ],
    )
    def k(table_hbm, idx_hbm, out_hbm, idx_v, rows_v0, rows_v1, out_v,
          sem0, sem1):
        wid = lax.axis_index("s") * NC + lax.axis_index("c")
        base_pair = wid * PAIRS_PER_W
        sems = (sem0, sem1)
        rows = (rows_v0, rows_v1)

        def reduce_pair(buf, p):
            rv = rows[buf]
            for half in range(2):
                r0 = half * SEQ
                for q in range(EMB // 16):
                    sl = pl.ds(q * 16, 16)
                    a0 = rv[r0, sl]
                    a1 = rv[r0 + 1, sl]
                    for j in range(2, SEQ, 2):
                        a0 = a0 + rv[r0 + j, sl]
                        a1 = a1 + rv[r0 + j + 1, sl]
                    out_v[2 * p + half, sl] = (a0 + a1) * _INV_SEQ

        def gather(p, buf):
            pltpu.async_copy(table_hbm.at[idx_v.at[p]], rows[buf],
                             sems[buf])

        def wait(buf):
            pltpu.make_async_copy(table_hbm.at[idx_v.at[0]], rows[buf],
                                  sems[buf]).wait()

        @pl.loop(0, NCH)
        def _(ch):
            pair0 = base_pair + ch * CP
            pltpu.sync_copy(idx_hbm.at[pl.ds(pair0, CP), :], idx_v)
            gather(0, 0)

            @pl.loop(0, CP, step=2)
            def _(p):
                gather(p + 1, 1)
                wait(0)
                reduce_pair(0, p)

                @pl.when(p + 2 < CP)
                def _():
                    gather(p + 2, 0)

                wait(1)
                reduce_pair(1, p + 1)

            pltpu.sync_copy(out_v, out_hbm.at[pl.ds(2 * pair0, 2 * CP), :])

    return k(table_pad, idx_pairs)


_BB = 2048
_NB = BATCH // _BB


def _mlp_body(xp_ref, xh_ref, w1p_ref, w1h_ref, b1_ref, w2_ref, b2_ref,
              w3_ref, b3_ref, wp_ref, bp_ref, o_ref):
    dot = functools.partial(jnp.dot, preferred_element_type=jnp.float32,
                            precision=lax.Precision.HIGHEST)
    h = dot(xp_ref[...], w1p_ref[...])
    h = h + dot(xh_ref[...], w1h_ref[...]) + b1_ref[...]
    h = jnp.maximum(h, 0.0)
    h = jnp.maximum(dot(h, w2_ref[...]) + b2_ref[...], 0.0)
    h = jnp.maximum(dot(h, w3_ref[...]) + b3_ref[...], 0.0)
    o_ref[...] = dot(h, wp_ref[...]) + bp_ref[...]


def _mlp_tc(pooled, W1p, W1h, b1, W2, b2, W3, b3, Wp, bp):
    full = lambda shape: pl.BlockSpec(shape, lambda i: (0, 0))
    return pl.pallas_call(
        _mlp_body,
        grid=(_NB,),
        in_specs=[
            pl.BlockSpec((_BB, EMB), lambda i: (i, 0)),
            pl.BlockSpec((_BB, EMB), lambda i: (i + _NB, 0)),
            full((EMB, HID)),
            full((EMB, HID)),
            full((1, HID)),
            full((HID, HID)),
            full((1, HID)),
            full((HID, HID)),
            full((1, HID)),
            full((HID, NCLS)),
            full((1, NCLS)),
        ],
        out_specs=pl.BlockSpec((_BB, NCLS), lambda i: (i, 0)),
        out_shape=jax.ShapeDtypeStruct((BATCH, NCLS), jnp.float32),
    )(pooled, pooled, W1p, W1h, b1, W2, b2, W3, b3, Wp, bp)


def kernel(premise, hypothesis, table, W1, b1, W2, b2, W3, b3, Wp, bp):
    table_pad = jnp.pad(table, ((0, 0), (0, EMB)))
    idx_pairs = jnp.concatenate([premise, hypothesis], axis=0)
    idx_pairs = idx_pairs.reshape(NPAIRS, 2 * SEQ)
    pooled = _pool_sc(table_pad, idx_pairs)
    return _mlp_tc(pooled, W1[:EMB], W1[EMB:], b1.reshape(1, HID),
                   W2, b2.reshape(1, HID), W3, b3.reshape(1, HID),
                   Wp, bp.reshape(1, NCLS))
